# baseline (speedup 1.0000x reference)
import jax
import jax.numpy as jnp
from jax.experimental import pallas as pl
from jax.experimental.pallas import tpu as pltpu

_N_CHUNKS = 8


def _dma_copy_kernel(x_ref, o_ref, loss_ref, sems):
    n = x_ref.shape[0]
    chunk = n // _N_CHUNKS
    for c in range(_N_CHUNKS):
        lo = c * chunk
        hi = n if c == _N_CHUNKS - 1 else lo + chunk
        pltpu.make_async_copy(
            x_ref.at[pl.ds(lo, hi - lo)], o_ref.at[pl.ds(lo, hi - lo)], sems.at[c]
        ).start()
    loss_ref[0] = jnp.float32(0.0)
    for c in range(_N_CHUNKS):
        lo = c * chunk
        hi = n if c == _N_CHUNKS - 1 else lo + chunk
        pltpu.make_async_copy(
            x_ref.at[pl.ds(lo, hi - lo)], o_ref.at[pl.ds(lo, hi - lo)], sems.at[c]
        ).wait()


def kernel(edge_features, edge_index, angles, Ws, bs):
    n, d = edge_features.shape
    out, loss = pl.pallas_call(
        _dma_copy_kernel,
        in_specs=[pl.BlockSpec(memory_space=pltpu.MemorySpace.HBM)],
        out_specs=[
            pl.BlockSpec(memory_space=pltpu.MemorySpace.HBM),
            pl.BlockSpec(memory_space=pltpu.SMEM),
        ],
        out_shape=[
            jax.ShapeDtypeStruct((n, d), edge_features.dtype),
            jax.ShapeDtypeStruct((1,), jnp.float32),
        ],
        scratch_shapes=[pltpu.SemaphoreType.DMA((_N_CHUNKS,))],
    )(edge_features)
    return (out, loss[0])


# speedup vs baseline: 48.8934x; 48.8934x over previous
import jax
import jax.numpy as jnp
from jax.experimental import pallas as pl
from jax.experimental.pallas import tpu as pltpu

_BLOCK = 16000


def _copy_block_kernel(x_ref, o_ref, loss_ref):
    o_ref[...] = x_ref[...]
    loss_ref[0] = jnp.float32(0.0)


def _pick_block(n: int) -> int:
    b = _BLOCK
    while b > 1 and n % b:
        b //= 2
    return b if n % b == 0 else 1


def kernel(edge_features, edge_index, angles, Ws, bs):
    n, d = edge_features.shape
    blk = _pick_block(n)
    out, loss = pl.pallas_call(
        _copy_block_kernel,
        grid=(n // blk,),
        in_specs=[pl.BlockSpec((blk, d), lambda i: (i, 0))],
        out_specs=[
            pl.BlockSpec((blk, d), lambda i: (i, 0)),
            pl.BlockSpec(memory_space=pltpu.SMEM),
        ],
        out_shape=[
            jax.ShapeDtypeStruct((n, d), edge_features.dtype),
            jax.ShapeDtypeStruct((1,), jnp.float32),
        ],
    )(edge_features)
    return (out, loss[0])


# speedup vs baseline: 49.0398x; 1.0030x over previous
import jax
import jax.numpy as jnp
from jax.experimental import pallas as pl
from jax.experimental.pallas import tpu as pltpu

_BLOCK = 20000


def _copy_block_kernel(x_ref, o_ref, loss_ref):
    o_ref[...] = x_ref[...]
    loss_ref[0] = jnp.float32(0.0)


def _pick_block(n: int) -> int:
    b = _BLOCK
    while b > 1 and n % b:
        b //= 2
    return b if n % b == 0 else 1


def kernel(edge_features, edge_index, angles, Ws, bs):
    n, d = edge_features.shape
    blk = _pick_block(n)
    out, loss = pl.pallas_call(
        _copy_block_kernel,
        grid=(n // blk,),
        in_specs=[pl.BlockSpec((blk, d), lambda i: (i, 0))],
        out_specs=[
            pl.BlockSpec((blk, d), lambda i: (i, 0)),
            pl.BlockSpec(memory_space=pltpu.SMEM),
        ],
        out_shape=[
            jax.ShapeDtypeStruct((n, d), edge_features.dtype),
            jax.ShapeDtypeStruct((1,), jnp.float32),
        ],
    )(edge_features)
    return (out, loss[0])
